# baseline (speedup 1.0000x reference)
import functools

import jax
import jax.numpy as jnp
from jax import lax
from jax.experimental import pallas as pl
from jax.experimental.pallas import tpu as pltpu
from jax.experimental.pallas import tpu_sc as plsc

_TOP_K = 8
_NUM_WORKERS = 32
_LANES = 16


def _logits_body(x_ref, w_ref, b_ref, logits_ref):
    logits_ref[...] = (
        jnp.dot(x_ref[...], w_ref[...], preferred_element_type=jnp.float32)
        + b_ref[...]
    )


def _fused_body(x_ref, w_ref, b_ref, logits_ref, weights_ref, experts_ref):
    logits = (
        jnp.dot(x_ref[...], w_ref[...], preferred_element_type=jnp.float32)
        + b_ref[...]
    )
    logits_ref[...] = logits
    num_experts = logits.shape[-1]
    iota = jax.lax.broadcasted_iota(jnp.int32, logits.shape, 1)
    work = logits
    vals = []
    idxs = []
    for _ in range(_TOP_K):
        m = jnp.max(work, axis=1, keepdims=True)
        idx = jnp.min(jnp.where(work == m, iota, num_experts),
                      axis=1, keepdims=True)
        vals.append(m)
        idxs.append(idx)
        work = jnp.where(iota == idx, -jnp.inf, work)
    topv = jnp.concatenate(vals, axis=1)
    topi = jnp.concatenate(idxs, axis=1)
    e = jnp.exp(topv - topv[:, :1])
    weights_ref[...] = e / jnp.sum(e, axis=1, keepdims=True)
    experts_ref[...] = topi


def _tc_fused(x, w_t, bias, tb):
    num_tokens, hidden_dim = x.shape
    num_experts = w_t.shape[1]
    return pl.pallas_call(
        _fused_body,
        grid=(num_tokens // tb,),
        in_specs=[
            pl.BlockSpec((tb, hidden_dim), lambda i: (i, 0)),
            pl.BlockSpec((hidden_dim, num_experts), lambda i: (0, 0)),
            pl.BlockSpec((1, num_experts), lambda i: (0, 0)),
        ],
        out_specs=[
            pl.BlockSpec((tb, num_experts), lambda i: (i, 0)),
            pl.BlockSpec((tb, _TOP_K), lambda i: (i, 0)),
            pl.BlockSpec((tb, _TOP_K), lambda i: (i, 0)),
        ],
        out_shape=[
            jax.ShapeDtypeStruct((num_tokens, num_experts), jnp.float32),
            jax.ShapeDtypeStruct((num_tokens, _TOP_K), jnp.float32),
            jax.ShapeDtypeStruct((num_tokens, _TOP_K), jnp.int32),
        ],
        compiler_params=pltpu.CompilerParams(
            dimension_semantics=("arbitrary",),
        ),
    )(x, w_t, bias)


def _tc_logits(x, w_t, bias, tb):
    num_tokens, hidden_dim = x.shape
    num_experts = w_t.shape[1]
    return pl.pallas_call(
        _logits_body,
        grid=(num_tokens // tb,),
        in_specs=[
            pl.BlockSpec((tb, hidden_dim), lambda i: (i, 0)),
            pl.BlockSpec((hidden_dim, num_experts), lambda i: (0, 0)),
            pl.BlockSpec((1, num_experts), lambda i: (0, 0)),
        ],
        out_specs=pl.BlockSpec((tb, num_experts), lambda i: (i, 0)),
        out_shape=jax.ShapeDtypeStruct((num_tokens, num_experts), jnp.float32),
        compiler_params=pltpu.CompilerParams(
            dimension_semantics=("arbitrary",),
        ),
    )(x, w_t, bias)


def _sc_topk(logits, dep):
    num_tokens, num_experts = logits.shape
    per = num_tokens // _NUM_WORKERS
    groups = per // _LANES

    @functools.partial(
        pl.kernel,
        out_type=[
            jax.ShapeDtypeStruct((num_tokens * _TOP_K,), jnp.float32),
            jax.ShapeDtypeStruct((num_tokens * _TOP_K,), jnp.int32),
        ],
        mesh=plsc.VectorSubcoreMesh(
            core_axis_name="c", subcore_axis_name="s"
        ),
        scratch_types=[
            pltpu.VMEM((per * num_experts,), jnp.float32),
            pltpu.VMEM((per * _TOP_K,), jnp.float32),
            pltpu.VMEM((per * _TOP_K,), jnp.int32),
        ],
        compiler_params=pltpu.CompilerParams(
            needs_layout_passes=False,
            skip_device_barrier=True,
        ),
    )
    def run(logits_hbm, dep_hbm, w_hbm, e_hbm, lt, wv, ev):
        del dep_hbm
        wid = lax.axis_index("s") * 2 + lax.axis_index("c")
        base = wid * per
        pltpu.sync_copy(
            logits_hbm.at[pl.ds(base * num_experts, per * num_experts)], lt
        )
        lane = lax.iota(jnp.int32, _LANES)
        neg = jnp.full((_LANES,), -jnp.inf, jnp.float32)
        zero_i = jnp.zeros((_LANES,), jnp.int32)
        one_i = jnp.full((_LANES,), 1, jnp.int32)
        nblk = num_experts // 8

        def do_group(g):
            tok = g * _LANES + lane
            tok_e = tok * num_experts
            tok_k = tok * _TOP_K

            bv = []
            bi = []
            for b in range(nblk):
                cv = plsc.load_gather(lt, [tok_e + (b * 8)])
                ci = jnp.full((_LANES,), b * 8, jnp.int32)
                for j in range(1, 8):
                    e = b * 8 + j
                    v = plsc.load_gather(lt, [tok_e + e])
                    m = v > cv
                    cv = jnp.where(m, v, cv)
                    ci = jnp.where(m, jnp.full((_LANES,), e, jnp.int32), ci)
                bv.append(cv)
                bi.append(ci)

            sel_lo = zero_i
            sel_hi = zero_i
            topv = []
            topi = []
            for kk in range(_TOP_K):
                tv = list(bv)
                ti = list(bi)
                while len(tv) > 1:
                    nv = []
                    ni = []
                    for p in range(0, len(tv), 2):
                        m = tv[p + 1] > tv[p]
                        nv.append(jnp.where(m, tv[p + 1], tv[p]))
                        ni.append(jnp.where(m, ti[p + 1], ti[p]))
                    tv = nv
                    ti = ni
                best_v = tv[0]
                best_i = ti[0]
                topv.append(best_v)
                topi.append(best_i)
                if kk == _TOP_K - 1:
                    break
                is_lo = best_i < 32
                bit = jnp.left_shift(one_i, best_i & 31)
                sel_lo = sel_lo | jnp.where(is_lo, bit, zero_i)
                sel_hi = sel_hi | jnp.where(is_lo, zero_i, bit)
                pos0 = best_i & (~7)
                word = jnp.where(pos0 < 32, sel_lo, sel_hi)
                sh0 = pos0 & 31
                nbv = neg
                nbi = zero_i
                for j in range(8):
                    pos = pos0 + j
                    v = plsc.load_gather(lt, [tok_e + pos])
                    b_ = jnp.right_shift(word, sh0 + j) & one_i
                    v = jnp.where(b_ == 1, neg, v)
                    m = v > nbv
                    nbv = jnp.where(m, v, nbv)
                    nbi = jnp.where(m, pos, nbi)
                for b in range(nblk):
                    mb = pos0 == (b * 8)
                    bv[b] = jnp.where(mb, nbv, bv[b])
                    bi[b] = jnp.where(mb, nbi, bi[b])

            mx = topv[0]
            exps = [jnp.exp(v - mx) for v in topv]
            tot = exps[0]
            for x_ in exps[1:]:
                tot = tot + x_
            inv = 1.0 / tot
            for kk in range(_TOP_K):
                plsc.store_scatter(wv, [tok_k + kk], exps[kk] * inv)
                plsc.store_scatter(ev, [tok_k + kk], topi[kk])

        def group_body(g, carry):
            do_group(g * 2)
            do_group(g * 2 + 1)
            return carry

        lax.fori_loop(0, groups // 2, group_body, jnp.int32(0))
        pltpu.sync_copy(wv, w_hbm.at[pl.ds(base * _TOP_K, per * _TOP_K)])
        pltpu.sync_copy(ev, e_hbm.at[pl.ds(base * _TOP_K, per * _TOP_K)])

    w_flat, e_flat = run(logits.reshape(-1), dep)
    return (
        w_flat.reshape(num_tokens, _TOP_K),
        e_flat.reshape(num_tokens, _TOP_K),
    )


@jax.jit
def kernel(hidden_states, gate_w, gate_b):
    num_tokens, hidden_dim = hidden_states.shape
    num_experts = gate_w.shape[0]
    w_t = gate_w.T
    bias = gate_b.reshape(1, num_experts)

    tb = 1024
    while num_tokens % tb:
        tb //= 2

    slab = _NUM_WORKERS * 2 * _LANES
    sc_tokens = (num_tokens * 3 // 4) // slab * slab
    if num_tokens % tb or sc_tokens % tb:
        sc_tokens = 0

    if sc_tokens == 0:
        return _tc_fused(hidden_states, w_t, bias, tb)

    x_sc = lax.slice_in_dim(hidden_states, 0, sc_tokens)
    x_tc = lax.slice_in_dim(hidden_states, sc_tokens, num_tokens)
    logits_sc = _tc_logits(x_sc, w_t, bias, tb)
    dep = jnp.zeros((_LANES,), jnp.float32)
    w_sc, e_sc = _sc_topk(logits_sc, dep)
    logits_tc, w_tc, e_tc = _tc_fused(x_tc, w_t, bias, tb)
    return (
        jnp.concatenate([logits_sc, logits_tc], 0),
        jnp.concatenate([w_sc, w_tc], 0),
        jnp.concatenate([e_sc, e_tc], 0),
    )


# speedup vs baseline: 2.1721x; 2.1721x over previous
import functools

import jax
import jax.numpy as jnp
from jax import lax
from jax.experimental import pallas as pl
from jax.experimental.pallas import tpu as pltpu
from jax.experimental.pallas import tpu_sc as plsc

_TOP_K = 8
_NUM_WORKERS = 32
_LANES = 16


def _logits_body(x_ref, w_ref, b_ref, logits_ref):
    logits_ref[...] = (
        jnp.dot(x_ref[...], w_ref[...], preferred_element_type=jnp.float32)
        + b_ref[...]
    )


def _fused_body(x_ref, w_ref, b_ref, logits_ref, weights_ref, experts_ref):
    logits = (
        jnp.dot(x_ref[...], w_ref[...], preferred_element_type=jnp.float32)
        + b_ref[...]
    )
    logits_ref[...] = logits
    num_experts = logits.shape[-1]
    iota = jax.lax.broadcasted_iota(jnp.int32, logits.shape, 1)
    work = logits
    vals = []
    idxs = []
    for _ in range(_TOP_K):
        m = jnp.max(work, axis=1, keepdims=True)
        idx = jnp.min(jnp.where(work == m, iota, num_experts),
                      axis=1, keepdims=True)
        vals.append(m)
        idxs.append(idx)
        work = jnp.where(iota == idx, -jnp.inf, work)
    topv = jnp.concatenate(vals, axis=1)
    topi = jnp.concatenate(idxs, axis=1)
    e = jnp.exp(topv - topv[:, :1])
    weights_ref[...] = e / jnp.sum(e, axis=1, keepdims=True)
    experts_ref[...] = topi


def _tc_fused(x, w_t, bias, tb, off_blocks, out_tokens):
    hidden_dim = x.shape[1]
    num_experts = w_t.shape[1]
    return pl.pallas_call(
        _fused_body,
        grid=(out_tokens // tb,),
        in_specs=[
            pl.BlockSpec((tb, hidden_dim), lambda i: (i + off_blocks, 0)),
            pl.BlockSpec((hidden_dim, num_experts), lambda i: (0, 0)),
            pl.BlockSpec((1, num_experts), lambda i: (0, 0)),
        ],
        out_specs=[
            pl.BlockSpec((tb, num_experts), lambda i: (i, 0)),
            pl.BlockSpec((tb, _TOP_K), lambda i: (i, 0)),
            pl.BlockSpec((tb, _TOP_K), lambda i: (i, 0)),
        ],
        out_shape=[
            jax.ShapeDtypeStruct((out_tokens, num_experts), jnp.float32),
            jax.ShapeDtypeStruct((out_tokens, _TOP_K), jnp.float32),
            jax.ShapeDtypeStruct((out_tokens, _TOP_K), jnp.int32),
        ],
        compiler_params=pltpu.CompilerParams(
            dimension_semantics=("arbitrary",),
        ),
    )(x, w_t, bias)


def _tc_logits(x, w_t, bias, tb, off_blocks, out_tokens):
    hidden_dim = x.shape[1]
    num_experts = w_t.shape[1]
    return pl.pallas_call(
        _logits_body,
        grid=(out_tokens // tb,),
        in_specs=[
            pl.BlockSpec((tb, hidden_dim), lambda i: (i + off_blocks, 0)),
            pl.BlockSpec((hidden_dim, num_experts), lambda i: (0, 0)),
            pl.BlockSpec((1, num_experts), lambda i: (0, 0)),
        ],
        out_specs=pl.BlockSpec((tb, num_experts), lambda i: (i, 0)),
        out_shape=jax.ShapeDtypeStruct((out_tokens, num_experts), jnp.float32),
        compiler_params=pltpu.CompilerParams(
            dimension_semantics=("arbitrary",),
        ),
    )(x, w_t, bias)


def _sc_topk(logits, dep):
    num_tokens, num_experts = logits.shape
    per = num_tokens // _NUM_WORKERS
    groups = per // _LANES

    @functools.partial(
        pl.kernel,
        out_type=[
            jax.ShapeDtypeStruct((num_tokens * _TOP_K,), jnp.float32),
            jax.ShapeDtypeStruct((num_tokens * _TOP_K,), jnp.int32),
        ],
        mesh=plsc.VectorSubcoreMesh(
            core_axis_name="c", subcore_axis_name="s"
        ),
        scratch_types=[
            pltpu.VMEM((per * num_experts,), jnp.float32),
            pltpu.VMEM((per * _TOP_K,), jnp.float32),
            pltpu.VMEM((per * _TOP_K,), jnp.int32),
        ],
        compiler_params=pltpu.CompilerParams(
            needs_layout_passes=False,
            skip_device_barrier=True,
        ),
    )
    def run(logits_hbm, dep_hbm, w_hbm, e_hbm, lt, wv, ev):
        del dep_hbm
        wid = lax.axis_index("s") * 2 + lax.axis_index("c")
        base = wid * per
        pltpu.sync_copy(
            logits_hbm.at[pl.ds(base * num_experts, per * num_experts)], lt
        )
        lane = lax.iota(jnp.int32, _LANES)
        neg = jnp.full((_LANES,), -jnp.inf, jnp.float32)
        zero_i = jnp.zeros((_LANES,), jnp.int32)
        one_i = jnp.full((_LANES,), 1, jnp.int32)
        nblk = num_experts // 8

        def do_group(g):
            tok = g * _LANES + lane
            tok_e = tok * num_experts
            tok_k = tok * _TOP_K

            bv = []
            bi = []
            for b in range(nblk):
                cv = plsc.load_gather(lt, [tok_e + (b * 8)])
                ci = jnp.full((_LANES,), b * 8, jnp.int32)
                for j in range(1, 8):
                    e = b * 8 + j
                    v = plsc.load_gather(lt, [tok_e + e])
                    m = v > cv
                    cv = jnp.where(m, v, cv)
                    ci = jnp.where(m, jnp.full((_LANES,), e, jnp.int32), ci)
                bv.append(cv)
                bi.append(ci)

            sel_lo = zero_i
            sel_hi = zero_i
            topv = []
            topi = []
            for kk in range(_TOP_K):
                tv = list(bv)
                ti = list(bi)
                while len(tv) > 1:
                    nv = []
                    ni = []
                    for p in range(0, len(tv), 2):
                        m = tv[p + 1] > tv[p]
                        nv.append(jnp.where(m, tv[p + 1], tv[p]))
                        ni.append(jnp.where(m, ti[p + 1], ti[p]))
                    tv = nv
                    ti = ni
                best_v = tv[0]
                best_i = ti[0]
                topv.append(best_v)
                topi.append(best_i)
                if kk == _TOP_K - 1:
                    break
                is_lo = best_i < 32
                bit = jnp.left_shift(one_i, best_i & 31)
                sel_lo = sel_lo | jnp.where(is_lo, bit, zero_i)
                sel_hi = sel_hi | jnp.where(is_lo, zero_i, bit)
                pos0 = best_i & (~7)
                word = jnp.where(pos0 < 32, sel_lo, sel_hi)
                sh0 = pos0 & 31
                nbv = neg
                nbi = zero_i
                for j in range(8):
                    pos = pos0 + j
                    v = plsc.load_gather(lt, [tok_e + pos])
                    b_ = jnp.right_shift(word, sh0 + j) & one_i
                    v = jnp.where(b_ == 1, neg, v)
                    m = v > nbv
                    nbv = jnp.where(m, v, nbv)
                    nbi = jnp.where(m, pos, nbi)
                for b in range(nblk):
                    mb = pos0 == (b * 8)
                    bv[b] = jnp.where(mb, nbv, bv[b])
                    bi[b] = jnp.where(mb, nbi, bi[b])

            mx = topv[0]
            exps = [jnp.exp(v - mx) for v in topv]
            tot = exps[0]
            for x_ in exps[1:]:
                tot = tot + x_
            inv = 1.0 / tot
            for kk in range(_TOP_K):
                plsc.store_scatter(wv, [tok_k + kk], exps[kk] * inv)
                plsc.store_scatter(ev, [tok_k + kk], topi[kk])

        def group_body(g, carry):
            do_group(g * 2)
            do_group(g * 2 + 1)
            return carry

        lax.fori_loop(0, groups // 2, group_body, jnp.int32(0))
        pltpu.sync_copy(wv, w_hbm.at[pl.ds(base * _TOP_K, per * _TOP_K)])
        pltpu.sync_copy(ev, e_hbm.at[pl.ds(base * _TOP_K, per * _TOP_K)])

    w_flat, e_flat = run(logits.reshape(-1), dep)
    return (
        w_flat.reshape(num_tokens, _TOP_K),
        e_flat.reshape(num_tokens, _TOP_K),
    )


@jax.jit
def kernel(hidden_states, gate_w, gate_b):
    num_tokens, hidden_dim = hidden_states.shape
    num_experts = gate_w.shape[0]
    w_t = gate_w.T
    bias = gate_b.reshape(1, num_experts)

    tb = 1024
    while num_tokens % tb:
        tb //= 2

    slab = _NUM_WORKERS * 2 * _LANES
    sc_tokens = (num_tokens * 3 // 4) // slab * slab
    if num_tokens % tb or sc_tokens % tb:
        sc_tokens = 0

    if sc_tokens == 0:
        return _tc_fused(hidden_states, w_t, bias, tb, 0, num_tokens)

    logits_sc = _tc_logits(hidden_states, w_t, bias, tb, 0, sc_tokens)
    dep = jnp.zeros((_LANES,), jnp.float32)
    w_sc, e_sc = _sc_topk(logits_sc, dep)
    logits_tc, w_tc, e_tc = _tc_fused(
        hidden_states, w_t, bias, tb, sc_tokens // tb,
        num_tokens - sc_tokens,
    )
    return (
        jnp.concatenate([logits_sc, logits_tc], 0),
        jnp.concatenate([w_sc, w_tc], 0),
        jnp.concatenate([e_sc, e_tc], 0),
    )


# speedup vs baseline: 2.2035x; 1.0145x over previous
import functools

import jax
import jax.numpy as jnp
from jax import lax
from jax.experimental import pallas as pl
from jax.experimental.pallas import tpu as pltpu
from jax.experimental.pallas import tpu_sc as plsc

_TOP_K = 8
_NUM_WORKERS = 32
_LANES = 16


def _logits_body(x_ref, w_ref, b_ref, logits_ref):
    logits_ref[...] = (
        jnp.dot(x_ref[...], w_ref[...], preferred_element_type=jnp.float32)
        + b_ref[...]
    )


def _fused_body(x_ref, w_ref, b_ref, logits_ref, weights_ref, experts_ref):
    logits = (
        jnp.dot(x_ref[...], w_ref[...], preferred_element_type=jnp.float32)
        + b_ref[...]
    )
    logits_ref[...] = logits
    num_experts = logits.shape[-1]
    iota = jax.lax.broadcasted_iota(jnp.int32, logits.shape, 1)
    work = logits
    vals = []
    idxs = []
    for _ in range(_TOP_K):
        m = jnp.max(work, axis=1, keepdims=True)
        idx = jnp.min(jnp.where(work == m, iota, num_experts),
                      axis=1, keepdims=True)
        vals.append(m)
        idxs.append(idx)
        work = jnp.where(iota == idx, -jnp.inf, work)
    topv = jnp.concatenate(vals, axis=1)
    topi = jnp.concatenate(idxs, axis=1)
    e = jnp.exp(topv - topv[:, :1])
    weights_ref[...] = e / jnp.sum(e, axis=1, keepdims=True)
    experts_ref[...] = topi


def _tc_fused(x, w_t, bias, tb, off_blocks, out_tokens):
    hidden_dim = x.shape[1]
    num_experts = w_t.shape[1]
    return pl.pallas_call(
        _fused_body,
        grid=(out_tokens // tb,),
        in_specs=[
            pl.BlockSpec((tb, hidden_dim), lambda i: (i + off_blocks, 0)),
            pl.BlockSpec((hidden_dim, num_experts), lambda i: (0, 0)),
            pl.BlockSpec((1, num_experts), lambda i: (0, 0)),
        ],
        out_specs=[
            pl.BlockSpec((tb, num_experts), lambda i: (i, 0)),
            pl.BlockSpec((tb, _TOP_K), lambda i: (i, 0)),
            pl.BlockSpec((tb, _TOP_K), lambda i: (i, 0)),
        ],
        out_shape=[
            jax.ShapeDtypeStruct((out_tokens, num_experts), jnp.float32),
            jax.ShapeDtypeStruct((out_tokens, _TOP_K), jnp.float32),
            jax.ShapeDtypeStruct((out_tokens, _TOP_K), jnp.int32),
        ],
        compiler_params=pltpu.CompilerParams(
            dimension_semantics=("arbitrary",),
        ),
    )(x, w_t, bias)


def _tc_logits(x, w_t, bias, tb, off_blocks, out_tokens):
    hidden_dim = x.shape[1]
    num_experts = w_t.shape[1]
    return pl.pallas_call(
        _logits_body,
        grid=(out_tokens // tb,),
        in_specs=[
            pl.BlockSpec((tb, hidden_dim), lambda i: (i + off_blocks, 0)),
            pl.BlockSpec((hidden_dim, num_experts), lambda i: (0, 0)),
            pl.BlockSpec((1, num_experts), lambda i: (0, 0)),
        ],
        out_specs=pl.BlockSpec((tb, num_experts), lambda i: (i, 0)),
        out_shape=jax.ShapeDtypeStruct((out_tokens, num_experts), jnp.float32),
        compiler_params=pltpu.CompilerParams(
            dimension_semantics=("arbitrary",),
        ),
    )(x, w_t, bias)


def _sc_topk(logits, dep):
    num_tokens, num_experts = logits.shape
    per = num_tokens // _NUM_WORKERS
    groups = per // _LANES

    @functools.partial(
        pl.kernel,
        out_type=[
            jax.ShapeDtypeStruct((num_tokens * _TOP_K,), jnp.float32),
            jax.ShapeDtypeStruct((num_tokens * _TOP_K,), jnp.int32),
        ],
        mesh=plsc.VectorSubcoreMesh(
            core_axis_name="c", subcore_axis_name="s"
        ),
        scratch_types=[
            pltpu.VMEM((per * num_experts,), jnp.float32),
            pltpu.VMEM((per * _TOP_K,), jnp.float32),
            pltpu.VMEM((per * _TOP_K,), jnp.int32),
        ],
        compiler_params=pltpu.CompilerParams(
            needs_layout_passes=False,
            skip_device_barrier=True,
        ),
    )
    def run(logits_hbm, dep_hbm, w_hbm, e_hbm, lt, wv, ev):
        del dep_hbm
        wid = lax.axis_index("s") * 2 + lax.axis_index("c")
        base = wid * per
        pltpu.sync_copy(
            logits_hbm.at[pl.ds(base * num_experts, per * num_experts)], lt
        )
        lane = lax.iota(jnp.int32, _LANES)
        neg = jnp.full((_LANES,), -jnp.inf, jnp.float32)
        zero_i = jnp.zeros((_LANES,), jnp.int32)
        one_i = jnp.full((_LANES,), 1, jnp.int32)
        nblk = num_experts // 8

        def do_group(g):
            tok = g * _LANES + lane
            tok_e = tok * num_experts
            tok_k = tok * _TOP_K

            bv = []
            bi = []
            for b in range(nblk):
                cv = plsc.load_gather(lt, [tok_e + (b * 8)])
                ci = jnp.full((_LANES,), b * 8, jnp.int32)
                for j in range(1, 8):
                    e = b * 8 + j
                    v = plsc.load_gather(lt, [tok_e + e])
                    m = v > cv
                    cv = jnp.where(m, v, cv)
                    ci = jnp.where(m, jnp.full((_LANES,), e, jnp.int32), ci)
                bv.append(cv)
                bi.append(ci)

            sel_lo = zero_i
            sel_hi = zero_i
            topv = []
            topi = []
            for kk in range(_TOP_K):
                tv = list(bv)
                ti = list(bi)
                while len(tv) > 1:
                    nv = []
                    ni = []
                    for p in range(0, len(tv), 2):
                        m = tv[p + 1] > tv[p]
                        nv.append(jnp.where(m, tv[p + 1], tv[p]))
                        ni.append(jnp.where(m, ti[p + 1], ti[p]))
                    tv = nv
                    ti = ni
                best_v = tv[0]
                best_i = ti[0]
                topv.append(best_v)
                topi.append(best_i)
                if kk == _TOP_K - 1:
                    break
                is_lo = best_i < 32
                bit = jnp.left_shift(one_i, best_i & 31)
                sel_lo = sel_lo | jnp.where(is_lo, bit, zero_i)
                sel_hi = sel_hi | jnp.where(is_lo, zero_i, bit)
                pos0 = best_i & (~7)
                word = jnp.where(pos0 < 32, sel_lo, sel_hi)
                sh0 = pos0 & 31
                nbv = neg
                nbi = zero_i
                for j in range(8):
                    pos = pos0 + j
                    v = plsc.load_gather(lt, [tok_e + pos])
                    b_ = jnp.right_shift(word, sh0 + j) & one_i
                    v = jnp.where(b_ == 1, neg, v)
                    m = v > nbv
                    nbv = jnp.where(m, v, nbv)
                    nbi = jnp.where(m, pos, nbi)
                for b in range(nblk):
                    mb = pos0 == (b * 8)
                    bv[b] = jnp.where(mb, nbv, bv[b])
                    bi[b] = jnp.where(mb, nbi, bi[b])

            mx = topv[0]
            exps = [jnp.exp(v - mx) for v in topv]
            tot = exps[0]
            for x_ in exps[1:]:
                tot = tot + x_
            inv = 1.0 / tot
            for kk in range(_TOP_K):
                plsc.store_scatter(wv, [tok_k + kk], exps[kk] * inv)
                plsc.store_scatter(ev, [tok_k + kk], topi[kk])

        @plsc.parallel_loop(0, groups, 1, unroll=2)
        def _loop(g):
            do_group(g)
        pltpu.sync_copy(wv, w_hbm.at[pl.ds(base * _TOP_K, per * _TOP_K)])
        pltpu.sync_copy(ev, e_hbm.at[pl.ds(base * _TOP_K, per * _TOP_K)])

    w_flat, e_flat = run(logits.reshape(-1), dep)
    return (
        w_flat.reshape(num_tokens, _TOP_K),
        e_flat.reshape(num_tokens, _TOP_K),
    )


@jax.jit
def kernel(hidden_states, gate_w, gate_b):
    num_tokens, hidden_dim = hidden_states.shape
    num_experts = gate_w.shape[0]
    w_t = gate_w.T
    bias = gate_b.reshape(1, num_experts)

    tb = 1024
    while num_tokens % tb:
        tb //= 2

    slab = _NUM_WORKERS * 2 * _LANES
    sc_tokens = (num_tokens * 3 // 4) // slab * slab
    if num_tokens % tb or sc_tokens % tb:
        sc_tokens = 0

    if sc_tokens == 0:
        return _tc_fused(hidden_states, w_t, bias, tb, 0, num_tokens)

    logits_sc = _tc_logits(hidden_states, w_t, bias, tb, 0, sc_tokens)
    dep = jnp.zeros((_LANES,), jnp.float32)
    w_sc, e_sc = _sc_topk(logits_sc, dep)
    logits_tc, w_tc, e_tc = _tc_fused(
        hidden_states, w_t, bias, tb, sc_tokens // tb,
        num_tokens - sc_tokens,
    )
    return (
        jnp.concatenate([logits_sc, logits_tc], 0),
        jnp.concatenate([w_sc, w_tc], 0),
        jnp.concatenate([e_sc, e_tc], 0),
    )
